# baseline (speedup 1.0000x reference)
import functools

import jax
import jax.numpy as jnp
from jax import lax
from jax.experimental import pallas as pl
from jax.experimental.pallas import tpu as pltpu
from jax.experimental.pallas import tpu_sc as plsc

_H = 64
_P = 8
_NC = 2
_NS = 16


def _sc_gather_rows(table, idx, chunk):
    N, D = table.shape
    M = idx.shape[0]
    NW = _NC * _NS
    b_per_w = M // NW
    n_chunks = b_per_w // chunk
    mesh = plsc.VectorSubcoreMesh(core_axis_name="c", subcore_axis_name="s")

    @functools.partial(
        pl.kernel,
        out_type=jax.ShapeDtypeStruct((M, D), jnp.float32),
        mesh=mesh,
        scratch_types=[
            pltpu.VMEM((chunk,), jnp.int32),
            pltpu.VMEM((chunk, D), jnp.float32),
            pltpu.SemaphoreType.DMA,
        ],
    )
    def k(table_hbm, idx_hbm, out_hbm, idx_c, rows_v, sem):
        wid = lax.axis_index("s") * _NC + lax.axis_index("c")
        base = wid * b_per_w
        for c in range(n_chunks):
            off = base + c * chunk
            pltpu.sync_copy(idx_hbm.at[pl.ds(off, chunk)], idx_c)
            pltpu.async_copy(table_hbm.at[idx_c], rows_v, sem).wait()
            pltpu.sync_copy(rows_v, out_hbm.at[pl.ds(off, chunk)])

    return k(table, idx)


def _mlp_body(h_ref, gid_ref, mask_ref, w1_ref, b1_ref, w2_ref, b2_ref, o_ref):
    T = h_ref.shape[0]
    GH = w1_ref.shape[1]
    G = GH // _H

    x = h_ref[...].astype(jnp.bfloat16)
    h1 = jnp.dot(x, w1_ref[...], preferred_element_type=jnp.float32) + b1_ref[...]
    h1 = jnp.maximum(h1, 0.0)

    gid = gid_ref[...]
    lane_g = jax.lax.broadcasted_iota(jnp.int32, (T, GH), 1) // _H
    h1m = jnp.where(lane_g == gid, h1, 0.0)
    h1c = jnp.zeros((T, _H), jnp.float32)
    for g in range(G):
        h1c = h1c + h1m[:, g * _H : (g + 1) * _H]

    la = jnp.dot(h1c.astype(jnp.bfloat16), w2_ref[...],
                 preferred_element_type=jnp.float32)
    acc = jnp.zeros((T, _P), jnp.float32)
    for g in range(G):
        acc = acc + jnp.where(gid == g, la[:, g * _P : (g + 1) * _P] + b2_ref[g : g + 1, :], 0.0)

    logits = jnp.where(mask_ref[...] > 0, acc, -1e9)
    m = jnp.max(logits, axis=1, keepdims=True)
    e = jnp.exp(logits - m)
    o_ref[:, 0:_P] = e / jnp.sum(e, axis=1, keepdims=True)


def _mlp(h2, gid2, maskf, W1r, b1r, W2r, b2):
    B, D = h2.shape
    GH = W1r.shape[1]
    G = GH // _H
    T = 512
    return pl.pallas_call(
        _mlp_body,
        grid=(B // T,),
        in_specs=[
            pl.BlockSpec((T, D), lambda i: (i, 0)),
            pl.BlockSpec((T, 1), lambda i: (i, 0)),
            pl.BlockSpec((T, _P), lambda i: (i, 0)),
            pl.BlockSpec((D, GH), lambda i: (0, 0)),
            pl.BlockSpec((1, GH), lambda i: (0, 0)),
            pl.BlockSpec((_H, G * _P), lambda i: (0, 0)),
            pl.BlockSpec((G, _P), lambda i: (0, 0)),
        ],
        out_specs=pl.BlockSpec((T, 128), lambda i: (i, 0)),
        out_shape=jax.ShapeDtypeStruct((B, 128), jnp.float32),
    )(h2, gid2, maskf, W1r, b1r, W2r, b2)


def kernel(h_int, group_ids, feasible_mask, W1, b1, W2, b2):
    B, D = h_int.shape
    G, _, H = W1.shape
    P = W2.shape[2]

    W1r = W1.transpose(1, 0, 2).reshape(D, G * H).astype(jnp.bfloat16)
    b1r = b1.reshape(1, G * H)
    W2r = W2.transpose(1, 0, 2).reshape(H, G * P).astype(jnp.bfloat16)

    i = jnp.arange(B, dtype=jnp.int32)
    perm = (i * 5) % B
    inv = (i * 3277) % B

    gid2 = group_ids[perm].reshape(B, 1)
    maskf = feasible_mask[perm].astype(jnp.float32)

    h2 = _sc_gather_rows(h_int, perm, chunk=64)
    probs2 = _mlp(h2, gid2, maskf, W1r, b1r, W2r, b2)
    out128 = _sc_gather_rows(probs2, inv, chunk=256)
    return out128[:, :P]
